# baseline (speedup 1.0000x reference)
import functools

import jax
import jax.numpy as jnp
import numpy as np
from jax import lax
from jax.experimental import pallas as pl
from jax.experimental.pallas import tpu as pltpu
from jax.experimental.pallas import tpu_sc as plsc

_PRE_N = 100000
_POST_N = 100000
_N_EDGES = 6400000
_DT = 0.1
_TAU_AMPA = 2.0
_E_AMPA = 0.0
_G_BAR = 1.0
_DECAY = float(np.exp(-_DT / _TAU_AMPA))

_NC = 2
_NS = 16
_NW = _NC * _NS
_L = 16

_CH = 2048
_NCHUNK = _N_EDGES // _CH
_NJ = -(-(-(-_NCHUNK // _NW) + 1) // 3)

_PAD_N = 100352
_ROWS = _PAD_N // 128

_RCH = 25000
_NR = _POST_N // _RCH


def _sc_partial_g(snew, col2, val2, row2):
    mesh = plsc.VectorSubcoreMesh(core_axis_name="c", subcore_axis_name="s")

    @functools.partial(
        pl.kernel,
        mesh=mesh,
        out_type=[
            jax.ShapeDtypeStruct((_NW * _PAD_N,), jnp.float32),
            jax.ShapeDtypeStruct((_N_EDGES,), jnp.float32),
        ],
        compiler_params=pltpu.CompilerParams(needs_layout_passes=False),
        scratch_types=[
            pltpu.VMEM((_PRE_N,), jnp.float32),
            pltpu.VMEM((_CH,), jnp.int32),
            pltpu.VMEM((_CH,), jnp.int32),
            pltpu.VMEM((_CH,), jnp.int32),
            pltpu.VMEM((_CH,), jnp.float32),
            pltpu.VMEM((_CH,), jnp.float32),
            pltpu.VMEM((_CH,), jnp.float32),
            pltpu.VMEM((_CH,), jnp.int32),
            pltpu.VMEM((_CH,), jnp.int32),
            pltpu.VMEM((_CH,), jnp.int32),
            pltpu.VMEM((_CH,), jnp.float32),
            pltpu.VMEM((_CH,), jnp.float32),
            pltpu.VMEM((_CH,), jnp.float32),
            pltpu.SemaphoreType.DMA,
            pltpu.SemaphoreType.DMA,
            pltpu.SemaphoreType.DMA,
            pltpu.SemaphoreType.DMA,
            pltpu.SemaphoreType.DMA,
            pltpu.SemaphoreType.DMA,
            pltpu.SemaphoreType.DMA,
        ],
    )
    def kern(snew_hbm, col_hbm, val_hbm, row_hbm, out_hbm, con_hbm,
             sg_tile, col0, col1, col2, val0, val1, val2,
             row0, row1, row2, con0, con1, con2,
             lsem, dsem0, dsem1, dsem2, ssem0, ssem1, ssem2):
        cid = lax.axis_index("c")
        sid = lax.axis_index("s")
        wid = cid * _NS + sid
        colb = (col0, col1, col2)
        valb = (val0, val1, val2)
        rowb = (row0, row1, row2)
        conb = (con0, con1, con2)
        dsems = (dsem0, dsem1, dsem2)
        ssems = (ssem0, ssem1, ssem2)

        pltpu.async_copy(snew_hbm.at[pl.ds(0, _PRE_N)], s_tile := sg_tile,
                         lsem)
        pltpu.make_async_copy(snew_hbm.at[pl.ds(0, _PRE_N)], s_tile,
                              lsem).wait()

        def dma1_start(slot, c):
            pltpu.async_copy(col_hbm.at[c], colb[slot], dsems[slot])
            pltpu.async_copy(val_hbm.at[c], valb[slot], dsems[slot])

        def dma1_wait(slot, c):
            pltpu.make_async_copy(col_hbm.at[c], colb[slot],
                                  dsems[slot]).wait()
            pltpu.make_async_copy(val_hbm.at[c], valb[slot],
                                  dsems[slot]).wait()

        def out1_start(slot, c):
            pltpu.async_copy(conb[slot], con_hbm.at[pl.ds(c * _CH, _CH)],
                             ssems[slot])

        def out1_wait(slot):
            pltpu.make_async_copy(conb[slot], con_hbm.at[pl.ds(0, _CH)],
                                  ssems[slot]).wait()

        dma1_start(0, wid)

        def step1(j, b):
            k = 3 * j + b
            c = wid + _NW * k

            @pl.when(c < _NCHUNK)
            def _():
                nxt = (b + 1) % 3

                @pl.when(c + _NW < _NCHUNK)
                def _():
                    if b == 2:
                        out1_wait(nxt)
                    else:
                        @pl.when(j >= 1)
                        def _():
                            out1_wait(nxt)

                    dma1_start(nxt, c + _NW)

                dma1_wait(b, c)

                def gmul(i, _):
                    sl = pl.ds(i * _L, _L)
                    idx = colb[b][sl]
                    sv = plsc.load_gather(s_tile, [idx])
                    conb[b][sl] = valb[b][sl] * sv
                    return 0

                lax.fori_loop(0, _CH // _L, gmul, 0)
                out1_start(b, c)

        def body1(j, _):
            step1(j, 0)
            step1(j, 1)
            step1(j, 2)
            return 0

        lax.fori_loop(0, _NJ, body1, 0)

        for b in range(3):
            out1_wait(b)

        g_tile = sg_tile

        def zero_vec(i, _):
            g_tile[pl.ds(i * _L, _L)] = jnp.zeros((_L,), jnp.float32)
            return 0

        lax.fori_loop(0, _POST_N // _L, zero_vec, 0)

        def dma2_start(slot, c):
            pltpu.async_copy(row_hbm.at[c], rowb[slot], dsems[slot])
            pltpu.async_copy(con_hbm.at[pl.ds(c * _CH, _CH)], conb[slot],
                             dsems[slot])

        def dma2_wait(slot, c):
            pltpu.make_async_copy(row_hbm.at[c], rowb[slot],
                                  dsems[slot]).wait()
            pltpu.make_async_copy(con_hbm.at[pl.ds(c * _CH, _CH)],
                                  conb[slot], dsems[slot]).wait()

        dma2_start(0, wid)

        def step2(j, b):
            k = 3 * j + b
            c = wid + _NW * k

            @pl.when(c < _NCHUNK)
            def _():
                nxt = (b + 1) % 3

                @pl.when(c + _NW < _NCHUNK)
                def _():
                    dma2_start(nxt, c + _NW)

                dma2_wait(b, c)

                def scat(i, _):
                    sl = pl.ds(i * _L, _L)
                    idx = rowb[b][sl]
                    plsc.addupdate_scatter(g_tile, [idx], conb[b][sl])
                    return 0

                lax.fori_loop(0, _CH // _L, scat, 0)

        def body2(j, _):
            step2(j, 0)
            step2(j, 1)
            step2(j, 2)
            return 0

        lax.fori_loop(0, _NJ, body2, 0)

        for r in range(_NR):
            pltpu.sync_copy(g_tile.at[pl.ds(r * _RCH, _RCH)],
                            out_hbm.at[pl.ds(wid * _PAD_N + r * _RCH, _RCH)])

    return kern(snew, col2, val2, row2)[0]


def _tc_snew(pre_spikes, s):
    a = jnp.pad(s, (0, _PAD_N - _PRE_N)).reshape(_ROWS, 128)
    b = jnp.pad(pre_spikes, (0, _PAD_N - _PRE_N)).reshape(_ROWS, 128)

    def upd(s_ref, p_ref, o_ref):
        o_ref[...] = s_ref[...] * _DECAY + p_ref[...]

    out = pl.pallas_call(
        upd,
        out_shape=jax.ShapeDtypeStruct((_ROWS, 128), jnp.float32),
    )(a, b)
    return out.reshape(-1)


def _tc_finish(partial, post_v):
    a = partial.reshape(_NW, _ROWS, 128)
    b = jnp.pad(post_v, (0, _PAD_N - _POST_N)).reshape(_ROWS, 128)

    def fin(a_ref, b_ref, o_ref):
        g = a_ref[0]
        for i in range(1, _NW):
            g = g + a_ref[i]
        o_ref[...] = (_G_BAR * g) * (_E_AMPA - b_ref[...])

    out = pl.pallas_call(
        fin,
        out_shape=jax.ShapeDtypeStruct((_ROWS, 128), jnp.float32),
    )(a, b)
    return out.reshape(-1)[:_POST_N]


def kernel(pre_spikes, post_v, _row, _col, _val, s):
    col2 = _col.astype(jnp.int32).reshape(_NCHUNK, _CH)
    row2 = _row.astype(jnp.int32).reshape(_NCHUNK, _CH)
    val2 = _val.reshape(_NCHUNK, _CH)
    snew = _tc_snew(pre_spikes, s)
    partial = _sc_partial_g(snew, col2, val2, row2)
    return _tc_finish(partial, post_v)


# speedup vs baseline: 1.4562x; 1.4562x over previous
import functools

import jax
import jax.numpy as jnp
import numpy as np
from jax import lax
from jax.experimental import pallas as pl
from jax.experimental.pallas import tpu as pltpu
from jax.experimental.pallas import tpu_sc as plsc

_PRE_N = 100000
_POST_N = 100000
_N_EDGES = 6400000
_DT = 0.1
_TAU_AMPA = 2.0
_E_AMPA = 0.0
_G_BAR = 1.0
_DECAY = float(np.exp(-_DT / _TAU_AMPA))

_NC = 2
_NS = 16
_NW = _NC * _NS
_L = 16

_CH = 2048
_NCHUNK = _N_EDGES // _CH
_NJ = -(-(-(-_NCHUNK // _NW) + 1) // 3)

_PAD_N = 100352
_ROWS = _PAD_N // 128

_RCH = 25000
_NR = _POST_N // _RCH
_ZCH = 50000
_NP = _NC


def _sc_partial_g(snew, col2, val2, row2):
    mesh = plsc.VectorSubcoreMesh(core_axis_name="c", subcore_axis_name="s")

    @functools.partial(
        pl.kernel,
        mesh=mesh,
        out_type=jax.ShapeDtypeStruct((_NP * _PAD_N,), jnp.float32),
        compiler_params=pltpu.CompilerParams(needs_layout_passes=False),
        scratch_types=[
            pltpu.VMEM((_PRE_N,), jnp.float32),
            pltpu.VMEM_SHARED((_PRE_N,), jnp.float32),
            pltpu.VMEM((_CH,), jnp.int32),
            pltpu.VMEM((_CH,), jnp.int32),
            pltpu.VMEM((_CH,), jnp.int32),
            pltpu.VMEM((_CH,), jnp.float32),
            pltpu.VMEM((_CH,), jnp.float32),
            pltpu.VMEM((_CH,), jnp.float32),
            pltpu.VMEM((_CH,), jnp.int32),
            pltpu.VMEM((_CH,), jnp.int32),
            pltpu.VMEM((_CH,), jnp.int32),
            pltpu.VMEM((_CH,), jnp.float32),
            pltpu.VMEM((_CH,), jnp.float32),
            pltpu.VMEM((_CH,), jnp.float32),
            pltpu.SemaphoreType.DMA,
            pltpu.SemaphoreType.DMA,
            pltpu.SemaphoreType.DMA,
            pltpu.SemaphoreType.DMA,
            pltpu.SemaphoreType.DMA,
            pltpu.SemaphoreType.DMA,
            pltpu.SemaphoreType.DMA,
        ],
    )
    def kern(snew_hbm, col_hbm, val_hbm, row_hbm, out_hbm,
             s_tile, gsh, col0, col1, col2c, val0, val1, val2c,
             row0, row1, row2c, con0, con1, con2c,
             lsem, dsem0, dsem1, dsem2, ssem0, ssem1, ssem2):
        cid = lax.axis_index("c")
        sid = lax.axis_index("s")
        wid = cid * _NS + sid
        colb = (col0, col1, col2c)
        valb = (val0, val1, val2c)
        rowb = (row0, row1, row2c)
        conb = (con0, con1, con2c)
        dsems = (dsem0, dsem1, dsem2)
        ssems = (ssem0, ssem1, ssem2)

        def zero_vec(i, _):
            s_tile[pl.ds(i * _L, _L)] = jnp.zeros((_L,), jnp.float32)
            return 0

        lax.fori_loop(0, _ZCH // _L, zero_vec, 0)

        @pl.when(sid == 0)
        def _():
            for z in range(_PRE_N // _ZCH):
                pltpu.sync_copy(s_tile.at[pl.ds(0, _ZCH)],
                                gsh.at[pl.ds(z * _ZCH, _ZCH)])

        plsc.subcore_barrier()

        pltpu.async_copy(snew_hbm.at[pl.ds(0, _PRE_N)], s_tile, lsem)
        pltpu.make_async_copy(snew_hbm.at[pl.ds(0, _PRE_N)], s_tile,
                              lsem).wait()

        def dma_start(slot, c):
            pltpu.async_copy(col_hbm.at[c], colb[slot], dsems[slot])
            pltpu.async_copy(val_hbm.at[c], valb[slot], dsems[slot])
            pltpu.async_copy(row_hbm.at[c], rowb[slot], dsems[slot])

        def dma_wait(slot, c):
            pltpu.make_async_copy(col_hbm.at[c], colb[slot],
                                  dsems[slot]).wait()
            pltpu.make_async_copy(val_hbm.at[c], valb[slot],
                                  dsems[slot]).wait()
            pltpu.make_async_copy(row_hbm.at[c], rowb[slot],
                                  dsems[slot]).wait()

        def scat_start(slot):
            pltpu.async_copy(conb[slot], gsh.at[rowb[slot]], ssems[slot],
                             add=True)

        def scat_wait(slot):
            pltpu.make_async_copy(conb[slot], gsh.at[rowb[slot]],
                                  ssems[slot]).wait()

        dma_start(0, wid)

        def step(j, b):
            k = 3 * j + b
            c = wid + _NW * k

            @pl.when(c < _NCHUNK)
            def _():
                nxt = (b + 1) % 3

                @pl.when(c + _NW < _NCHUNK)
                def _():
                    if b == 2:
                        scat_wait(nxt)
                    else:
                        @pl.when(j >= 1)
                        def _():
                            scat_wait(nxt)

                    dma_start(nxt, c + _NW)

                dma_wait(b, c)

                def gmul(i, _):
                    sl = pl.ds(i * _L, _L)
                    idx = colb[b][sl]
                    sv = plsc.load_gather(s_tile, [idx])
                    conb[b][sl] = valb[b][sl] * sv
                    return 0

                lax.fori_loop(0, _CH // _L, gmul, 0)
                scat_start(b)

        def body(j, _):
            step(j, 0)
            step(j, 1)
            step(j, 2)
            return 0

        lax.fori_loop(0, _NJ, body, 0)

        for b in range(3):
            scat_wait(b)

        plsc.subcore_barrier()

        @pl.when(sid < _NR)
        def _():
            pltpu.sync_copy(gsh.at[pl.ds(sid * _RCH, _RCH)],
                            s_tile.at[pl.ds(0, _RCH)])
            pltpu.sync_copy(
                s_tile.at[pl.ds(0, _RCH)],
                out_hbm.at[pl.ds(cid * _PAD_N + sid * _RCH, _RCH)])

    return kern(snew, col2, val2, row2)


def _tc_snew(pre_spikes, s):
    a = jnp.pad(s, (0, _PAD_N - _PRE_N)).reshape(_ROWS, 128)
    b = jnp.pad(pre_spikes, (0, _PAD_N - _PRE_N)).reshape(_ROWS, 128)

    def upd(s_ref, p_ref, o_ref):
        o_ref[...] = s_ref[...] * _DECAY + p_ref[...]

    out = pl.pallas_call(
        upd,
        out_shape=jax.ShapeDtypeStruct((_ROWS, 128), jnp.float32),
    )(a, b)
    return out.reshape(-1)


def _tc_finish(partial, post_v):
    a = partial.reshape(_NP, _ROWS, 128)
    b = jnp.pad(post_v, (0, _PAD_N - _POST_N)).reshape(_ROWS, 128)

    def fin(a_ref, b_ref, o_ref):
        g = a_ref[0]
        for i in range(1, _NP):
            g = g + a_ref[i]
        o_ref[...] = (_G_BAR * g) * (_E_AMPA - b_ref[...])

    out = pl.pallas_call(
        fin,
        out_shape=jax.ShapeDtypeStruct((_ROWS, 128), jnp.float32),
    )(a, b)
    return out.reshape(-1)[:_POST_N]


def kernel(pre_spikes, post_v, _row, _col, _val, s):
    col2 = _col.astype(jnp.int32).reshape(_NCHUNK, _CH)
    row2 = _row.astype(jnp.int32).reshape(_NCHUNK, _CH)
    val2 = _val.reshape(_NCHUNK, _CH)
    snew = _tc_snew(pre_spikes, s)
    partial = _sc_partial_g(snew, col2, val2, row2)
    return _tc_finish(partial, post_v)


# speedup vs baseline: 2.2980x; 1.5782x over previous
import functools

import jax
import jax.numpy as jnp
import numpy as np
from jax import lax
from jax.experimental import pallas as pl
from jax.experimental.pallas import tpu as pltpu
from jax.experimental.pallas import tpu_sc as plsc

_PRE_N = 100000
_POST_N = 100000
_N_EDGES = 6400000
_DT = 0.1
_TAU_AMPA = 2.0
_E_AMPA = 0.0
_G_BAR = 1.0
_DECAY = float(np.exp(-_DT / _TAU_AMPA))

_NC = 2
_NS = 16
_NW = _NC * _NS
_L = 16

_CH = 2048
_NCHUNK = _N_EDGES // _CH
_NJ = -(-(-(-_NCHUNK // _NW) + 1) // 3)

_PAD_N = 100352
_ROWS = _PAD_N // 128

_RCH = 25000
_NR = _POST_N // _RCH
_ZCH = 50000
_NP = _NC


def _sc_partial_g(snew, col2, val2, row2):
    mesh = plsc.VectorSubcoreMesh(core_axis_name="c", subcore_axis_name="s")

    @functools.partial(
        pl.kernel,
        mesh=mesh,
        out_type=jax.ShapeDtypeStruct((_NP * _PAD_N,), jnp.float32),
        compiler_params=pltpu.CompilerParams(needs_layout_passes=False),
        scratch_types=[
            pltpu.VMEM((_PRE_N,), jnp.float32),
            pltpu.VMEM_SHARED((_PRE_N,), jnp.float32),
            pltpu.VMEM((_CH,), jnp.int32),
            pltpu.VMEM((_CH,), jnp.int32),
            pltpu.VMEM((_CH,), jnp.int32),
            pltpu.VMEM((_CH,), jnp.float32),
            pltpu.VMEM((_CH,), jnp.float32),
            pltpu.VMEM((_CH,), jnp.float32),
            pltpu.VMEM((_CH,), jnp.int32),
            pltpu.VMEM((_CH,), jnp.int32),
            pltpu.VMEM((_CH,), jnp.int32),
            pltpu.VMEM((_CH,), jnp.float32),
            pltpu.VMEM((_CH,), jnp.float32),
            pltpu.VMEM((_CH,), jnp.float32),
            pltpu.SemaphoreType.DMA,
            pltpu.SemaphoreType.DMA,
            pltpu.SemaphoreType.DMA,
            pltpu.SemaphoreType.DMA,
            pltpu.SemaphoreType.DMA,
            pltpu.SemaphoreType.DMA,
            pltpu.SemaphoreType.DMA,
        ],
    )
    def kern(snew_hbm, col_hbm, val_hbm, row_hbm, out_hbm,
             s_tile, gsh, col0, col1, col2c, val0, val1, val2c,
             row0, row1, row2c, con0, con1, con2c,
             lsem, dsem0, dsem1, dsem2, ssem0, ssem1, ssem2):
        cid = lax.axis_index("c")
        sid = lax.axis_index("s")
        wid = cid * _NS + sid
        colb = (col0, col1, col2c)
        valb = (val0, val1, val2c)
        rowb = (row0, row1, row2c)
        conb = (con0, con1, con2c)
        dsems = (dsem0, dsem1, dsem2)
        ssems = (ssem0, ssem1, ssem2)

        def zero_vec(i, _):
            s_tile[pl.ds(i * _L, _L)] = jnp.zeros((_L,), jnp.float32)
            return 0

        lax.fori_loop(0, _ZCH // _L, zero_vec, 0)

        @pl.when(sid == 0)
        def _():
            for z in range(_PRE_N // _ZCH):
                pltpu.sync_copy(s_tile.at[pl.ds(0, _ZCH)],
                                gsh.at[pl.ds(z * _ZCH, _ZCH)])

        plsc.subcore_barrier()

        pltpu.async_copy(snew_hbm.at[pl.ds(0, _PRE_N)], s_tile, lsem)
        pltpu.make_async_copy(snew_hbm.at[pl.ds(0, _PRE_N)], s_tile,
                              lsem).wait()

        def dma_start(slot, c):
            e = pl.ds(c * _CH, _CH)
            pltpu.async_copy(col_hbm.at[e], colb[slot], dsems[slot])
            pltpu.async_copy(val_hbm.at[e], valb[slot], dsems[slot])
            pltpu.async_copy(row_hbm.at[e], rowb[slot], dsems[slot])

        def dma_wait(slot, c):
            e = pl.ds(c * _CH, _CH)
            pltpu.make_async_copy(col_hbm.at[e], colb[slot],
                                  dsems[slot]).wait()
            pltpu.make_async_copy(val_hbm.at[e], valb[slot],
                                  dsems[slot]).wait()
            pltpu.make_async_copy(row_hbm.at[e], rowb[slot],
                                  dsems[slot]).wait()

        def scat_start(slot):
            pltpu.async_copy(conb[slot], gsh.at[rowb[slot]], ssems[slot],
                             add=True)

        def scat_wait(slot):
            pltpu.make_async_copy(conb[slot], gsh.at[rowb[slot]],
                                  ssems[slot]).wait()

        dma_start(0, wid)

        def step(j, b):
            k = 3 * j + b
            c = wid + _NW * k

            @pl.when(c < _NCHUNK)
            def _():
                nxt = (b + 1) % 3

                @pl.when(c + _NW < _NCHUNK)
                def _():
                    if b == 2:
                        scat_wait(nxt)
                    else:
                        @pl.when(j >= 1)
                        def _():
                            scat_wait(nxt)

                    dma_start(nxt, c + _NW)

                dma_wait(b, c)

                def gmul(i, _):
                    sl = pl.ds(i * _L, _L)
                    idx = colb[b][sl]
                    sv = plsc.load_gather(s_tile, [idx])
                    conb[b][sl] = valb[b][sl] * sv
                    return 0

                lax.fori_loop(0, _CH // _L, gmul, 0)
                scat_start(b)

        def body(j, _):
            step(j, 0)
            step(j, 1)
            step(j, 2)
            return 0

        lax.fori_loop(0, _NJ, body, 0)

        for b in range(3):
            scat_wait(b)

        plsc.subcore_barrier()

        @pl.when(sid < _NR)
        def _():
            pltpu.sync_copy(gsh.at[pl.ds(sid * _RCH, _RCH)],
                            s_tile.at[pl.ds(0, _RCH)])
            pltpu.sync_copy(
                s_tile.at[pl.ds(0, _RCH)],
                out_hbm.at[pl.ds(cid * _PAD_N + sid * _RCH, _RCH)])

    return kern(snew, col2, val2, row2)


def _tc_snew(pre_spikes, s):
    a = jnp.pad(s, (0, _PAD_N - _PRE_N)).reshape(_ROWS, 128)
    b = jnp.pad(pre_spikes, (0, _PAD_N - _PRE_N)).reshape(_ROWS, 128)

    def upd(s_ref, p_ref, o_ref):
        o_ref[...] = s_ref[...] * _DECAY + p_ref[...]

    out = pl.pallas_call(
        upd,
        out_shape=jax.ShapeDtypeStruct((_ROWS, 128), jnp.float32),
    )(a, b)
    return out.reshape(-1)


def _tc_finish(partial, post_v):
    a = partial.reshape(_NP, _ROWS, 128)
    b = jnp.pad(post_v, (0, _PAD_N - _POST_N)).reshape(_ROWS, 128)

    def fin(a_ref, b_ref, o_ref):
        g = a_ref[0]
        for i in range(1, _NP):
            g = g + a_ref[i]
        o_ref[...] = (_G_BAR * g) * (_E_AMPA - b_ref[...])

    out = pl.pallas_call(
        fin,
        out_shape=jax.ShapeDtypeStruct((_ROWS, 128), jnp.float32),
    )(a, b)
    return out.reshape(-1)[:_POST_N]


def kernel(pre_spikes, post_v, _row, _col, _val, s):
    col1 = _col.astype(jnp.int32)
    row1 = _row.astype(jnp.int32)
    snew = _tc_snew(pre_spikes, s)
    partial = _sc_partial_g(snew, col1, _val, row1)
    return _tc_finish(partial, post_v)


# speedup vs baseline: 2.3060x; 1.0035x over previous
import functools

import jax
import jax.numpy as jnp
import numpy as np
from jax import lax
from jax.experimental import pallas as pl
from jax.experimental.pallas import tpu as pltpu
from jax.experimental.pallas import tpu_sc as plsc

_PRE_N = 100000
_POST_N = 100000
_N_EDGES = 6400000
_DT = 0.1
_TAU_AMPA = 2.0
_E_AMPA = 0.0
_G_BAR = 1.0
_DECAY = float(np.exp(-_DT / _TAU_AMPA))

_NC = 2
_NS = 16
_NW = _NC * _NS
_L = 16

_CH = 2048
_NCHUNK = _N_EDGES // _CH
_NJ = -(-(-(-_NCHUNK // _NW) + 1) // 3)

_PAD_N = 100352
_ROWS = _PAD_N // 128

_RCH = 25000
_NR = _POST_N // _RCH
_ZCH = 50000
_NP = _NC
_UNROLL = 4


def _sc_partial_g(snew, col2, val2, row2):
    mesh = plsc.VectorSubcoreMesh(core_axis_name="c", subcore_axis_name="s")

    @functools.partial(
        pl.kernel,
        mesh=mesh,
        out_type=jax.ShapeDtypeStruct((_NP * _PAD_N,), jnp.float32),
        compiler_params=pltpu.CompilerParams(needs_layout_passes=False),
        scratch_types=[
            pltpu.VMEM((_PRE_N,), jnp.float32),
            pltpu.VMEM_SHARED((_PRE_N,), jnp.float32),
            pltpu.VMEM((_CH,), jnp.int32),
            pltpu.VMEM((_CH,), jnp.int32),
            pltpu.VMEM((_CH,), jnp.int32),
            pltpu.VMEM((_CH,), jnp.float32),
            pltpu.VMEM((_CH,), jnp.float32),
            pltpu.VMEM((_CH,), jnp.float32),
            pltpu.VMEM((_CH,), jnp.int32),
            pltpu.VMEM((_CH,), jnp.int32),
            pltpu.VMEM((_CH,), jnp.int32),
            pltpu.VMEM((_CH,), jnp.float32),
            pltpu.VMEM((_CH,), jnp.float32),
            pltpu.VMEM((_CH,), jnp.float32),
            pltpu.SemaphoreType.DMA,
            pltpu.SemaphoreType.DMA,
            pltpu.SemaphoreType.DMA,
            pltpu.SemaphoreType.DMA,
            pltpu.SemaphoreType.DMA,
            pltpu.SemaphoreType.DMA,
            pltpu.SemaphoreType.DMA,
        ],
    )
    def kern(snew_hbm, col_hbm, val_hbm, row_hbm, out_hbm,
             s_tile, gsh, col0, col1, col2c, val0, val1, val2c,
             row0, row1, row2c, con0, con1, con2c,
             lsem, dsem0, dsem1, dsem2, ssem0, ssem1, ssem2):
        cid = lax.axis_index("c")
        sid = lax.axis_index("s")
        wid = cid * _NS + sid
        colb = (col0, col1, col2c)
        valb = (val0, val1, val2c)
        rowb = (row0, row1, row2c)
        conb = (con0, con1, con2c)
        dsems = (dsem0, dsem1, dsem2)
        ssems = (ssem0, ssem1, ssem2)

        def zero_vec(i, _):
            s_tile[pl.ds(i * _L, _L)] = jnp.zeros((_L,), jnp.float32)
            return 0

        lax.fori_loop(0, _ZCH // _L, zero_vec, 0)

        @pl.when(sid == 0)
        def _():
            for z in range(_PRE_N // _ZCH):
                pltpu.sync_copy(s_tile.at[pl.ds(0, _ZCH)],
                                gsh.at[pl.ds(z * _ZCH, _ZCH)])

        plsc.subcore_barrier()

        pltpu.async_copy(snew_hbm.at[pl.ds(0, _PRE_N)], s_tile, lsem)
        pltpu.make_async_copy(snew_hbm.at[pl.ds(0, _PRE_N)], s_tile,
                              lsem).wait()

        def dma_start(slot, c):
            e = pl.ds(c * _CH, _CH)
            pltpu.async_copy(col_hbm.at[e], colb[slot], dsems[slot])
            pltpu.async_copy(val_hbm.at[e], valb[slot], dsems[slot])
            pltpu.async_copy(row_hbm.at[e], rowb[slot], dsems[slot])

        def dma_wait(slot, c):
            e = pl.ds(c * _CH, _CH)
            pltpu.make_async_copy(col_hbm.at[e], colb[slot],
                                  dsems[slot]).wait()
            pltpu.make_async_copy(val_hbm.at[e], valb[slot],
                                  dsems[slot]).wait()
            pltpu.make_async_copy(row_hbm.at[e], rowb[slot],
                                  dsems[slot]).wait()

        def scat_start(slot):
            pltpu.async_copy(conb[slot], gsh.at[rowb[slot]], ssems[slot],
                             add=True)

        def scat_wait(slot):
            pltpu.make_async_copy(conb[slot], gsh.at[rowb[slot]],
                                  ssems[slot]).wait()

        dma_start(0, wid)

        def step(j, b):
            k = 3 * j + b
            c = wid + _NW * k

            @pl.when(c < _NCHUNK)
            def _():
                nxt = (b + 1) % 3

                @pl.when(c + _NW < _NCHUNK)
                def _():
                    if b == 2:
                        scat_wait(nxt)
                    else:
                        @pl.when(j >= 1)
                        def _():
                            scat_wait(nxt)

                    dma_start(nxt, c + _NW)

                dma_wait(b, c)

                def gmul(i, _):
                    for u in range(_UNROLL):
                        sl = pl.ds(i * (_L * _UNROLL) + u * _L, _L)
                        idx = colb[b][sl]
                        sv = plsc.load_gather(s_tile, [idx])
                        conb[b][sl] = valb[b][sl] * sv
                    return 0

                lax.fori_loop(0, _CH // (_L * _UNROLL), gmul, 0)
                scat_start(b)

        def body(j, _):
            step(j, 0)
            step(j, 1)
            step(j, 2)
            return 0

        lax.fori_loop(0, _NJ, body, 0)

        for b in range(3):
            scat_wait(b)

        plsc.subcore_barrier()

        @pl.when(sid < _NR)
        def _():
            pltpu.sync_copy(gsh.at[pl.ds(sid * _RCH, _RCH)],
                            s_tile.at[pl.ds(0, _RCH)])
            pltpu.sync_copy(
                s_tile.at[pl.ds(0, _RCH)],
                out_hbm.at[pl.ds(cid * _PAD_N + sid * _RCH, _RCH)])

    return kern(snew, col2, val2, row2)


def _tc_snew(pre_spikes, s):
    a = jnp.pad(s, (0, _PAD_N - _PRE_N)).reshape(_ROWS, 128)
    b = jnp.pad(pre_spikes, (0, _PAD_N - _PRE_N)).reshape(_ROWS, 128)

    def upd(s_ref, p_ref, o_ref):
        o_ref[...] = s_ref[...] * _DECAY + p_ref[...]

    out = pl.pallas_call(
        upd,
        out_shape=jax.ShapeDtypeStruct((_ROWS, 128), jnp.float32),
    )(a, b)
    return out.reshape(-1)


def _tc_finish(partial, post_v):
    a = partial.reshape(_NP, _ROWS, 128)
    b = jnp.pad(post_v, (0, _PAD_N - _POST_N)).reshape(_ROWS, 128)

    def fin(a_ref, b_ref, o_ref):
        g = a_ref[0]
        for i in range(1, _NP):
            g = g + a_ref[i]
        o_ref[...] = (_G_BAR * g) * (_E_AMPA - b_ref[...])

    out = pl.pallas_call(
        fin,
        out_shape=jax.ShapeDtypeStruct((_ROWS, 128), jnp.float32),
    )(a, b)
    return out.reshape(-1)[:_POST_N]


def kernel(pre_spikes, post_v, _row, _col, _val, s):
    col1 = _col.astype(jnp.int32)
    row1 = _row.astype(jnp.int32)
    snew = _tc_snew(pre_spikes, s)
    partial = _sc_partial_g(snew, col1, _val, row1)
    return _tc_finish(partial, post_v)
